# baseline (speedup 1.0000x reference)
import functools
import math

import jax
import jax.numpy as jnp
from jax.experimental import pallas as pl
from jax.experimental.pallas import tpu as pltpu

LAMBDA_SUP = 10.0
LAMBDA_CON = 1.0
LAMBDA_AFF = 0.1
LAMBDA_DIST = 0.1
LAMBDA_BDY = 0.5
WARMUP_EPOCHS = 15
IGNORE_INDEX = 255

BLK = 4096


def _dense_loss_kernel(epoch_ref, sem_ref, geo_ref, tgt_ref, feat_ref,
                       proto_ref, aff_ref, d2s_ref, d2i_ref, bdy_ref,
                       out_ref, acc_ref):
    i = pl.program_id(0)
    nsteps = pl.num_programs(0)
    blk, NCLS = sem_ref.shape
    C = feat_ref.shape[1]
    K = aff_ref.shape[1]

    tgt = tgt_ref[...]
    valid = (tgt != IGNORE_INDEX)
    validf = valid.astype(jnp.float32)
    nvalid = jnp.sum(validf)
    cls_iota = jax.lax.broadcasted_iota(jnp.int32, (blk, NCLS), 1)
    onehot = (cls_iota == tgt).astype(jnp.float32)

    def softmax_parts(x):
        m = jnp.max(x, axis=1, keepdims=True)
        e = jnp.exp(x - m)
        s = jnp.sum(e, axis=1, keepdims=True)
        lse = jnp.log(s) + m
        p = e / s
        return p, lse

    sem = sem_ref[...]
    geo = geo_ref[...]
    p_sem, lse_sem = softmax_parts(sem)
    p_geo, lse_geo = softmax_parts(geo)
    nll_sem = (lse_sem[:, 0] - jnp.sum(sem * onehot, axis=1)) * validf[:, 0]
    nll_geo = (lse_geo[:, 0] - jnp.sum(geo * onehot, axis=1)) * validf[:, 0]
    nll_sum = jnp.sum(nll_sem) + jnp.sum(nll_geo)

    eps = 1e-6
    pse = p_sem + eps
    pge = p_geo + eps
    log_pse = jnp.log(pse)
    log_pge = jnp.log(pge)
    kl_sg = jnp.sum(pge * (log_pge - log_pse))
    kl_gs = jnp.sum(pse * (log_pse - log_pge))

    aff = aff_ref[...]
    d2s = d2s_ref[...]
    amask = (aff > 0.8).astype(jnp.float32)
    aff_num = jnp.sum(aff * d2s * amask) * (1.0 / math.sqrt(C))
    mask_sum = jnp.sum(amask)

    feat = feat_ref[...]
    fnorm = jnp.maximum(jnp.sqrt(jnp.sum(feat * feat, axis=1, keepdims=True)),
                        1e-12)
    nf = feat / fnorm
    proto = proto_ref[...]
    pnorm = jnp.maximum(jnp.sqrt(jnp.sum(proto * proto, axis=1,
                                         keepdims=True)), 1e-12)
    nproto = proto / pnorm
    sim = jax.lax.dot_general(nf, nproto, (((1,), (1,)), ((), ())),
                              preferred_element_type=jnp.float32)
    tsim = jnp.sum(sim * onehot, axis=1)
    dist_sum = jnp.sum(validf[:, 0] * (1.0 - tsim))

    d2i = d2i_ref[...]
    jd = jnp.sqrt(d2i)
    es = jnp.sum(jd, axis=1) * (1.0 / K)
    tb = jax.nn.sigmoid((es - 0.15) * 20.0)
    x = bdy_ref[...][:, 0]
    bce = jnp.maximum(x, 0.0) - x * tb + jnp.log1p(jnp.exp(-jnp.abs(x)))
    bce_sum = jnp.sum(bce)

    @pl.when(i == 0)
    def _init():
        for j in range(8):
            acc_ref[j] = 0.0

    acc_ref[0] += nll_sum
    acc_ref[1] += nvalid
    acc_ref[2] += kl_sg
    acc_ref[3] += kl_gs
    acc_ref[4] += aff_num
    acc_ref[5] += mask_sum
    acc_ref[6] += dist_sum
    acc_ref[7] += bce_sum

    @pl.when(i == nsteps - 1)
    def _finalize():
        BN = blk * nsteps
        nv = jnp.maximum(acc_ref[1], 1.0)
        loss_sup = acc_ref[0] / nv
        epoch = epoch_ref[0]
        in_warmup = epoch < WARMUP_EPOCHS
        progress = jnp.clip(
            (epoch.astype(jnp.float32) - 1.0) / WARMUP_EPOCHS, 0.0, 1.0)
        lam_con = jnp.where(in_warmup, LAMBDA_CON * progress * 0.1,
                            LAMBDA_CON)
        kl_sg_m = acc_ref[2] / BN
        kl_gs_m = acc_ref[3] / BN
        loss_con = jnp.where(in_warmup, kl_sg_m, (kl_sg_m + kl_gs_m) * 0.5)
        loss_aff = acc_ref[4] / (acc_ref[5] + 1e-6)
        loss_dist = acc_ref[6] / nv
        loss_bdy = acc_ref[7] / BN
        out_ref[0, 0] = (loss_sup * LAMBDA_SUP + loss_con * lam_con
                         + loss_aff * LAMBDA_AFF + loss_dist * LAMBDA_DIST
                         + loss_bdy * LAMBDA_BDY)


def _dense_loss(epoch_arr, sem_logits, geo_logits, target2d, feat_flat,
                prototypes, aff_flat, d2s, d2i, bdy_flat):
    BN, NCLS = sem_logits.shape
    K = aff_flat.shape[1]
    C = feat_flat.shape[1]
    nsteps = BN // BLK
    out = pl.pallas_call(
        _dense_loss_kernel,
        grid=(nsteps,),
        in_specs=[
            pl.BlockSpec(memory_space=pltpu.SMEM),
            pl.BlockSpec((BLK, NCLS), lambda i: (i, 0)),
            pl.BlockSpec((BLK, NCLS), lambda i: (i, 0)),
            pl.BlockSpec((BLK, 1), lambda i: (i, 0)),
            pl.BlockSpec((BLK, C), lambda i: (i, 0)),
            pl.BlockSpec((prototypes.shape[0], C), lambda i: (0, 0)),
            pl.BlockSpec((BLK, K), lambda i: (i, 0)),
            pl.BlockSpec((BLK, K), lambda i: (i, 0)),
            pl.BlockSpec((BLK, K), lambda i: (i, 0)),
            pl.BlockSpec((BLK, 1), lambda i: (i, 0)),
        ],
        out_specs=pl.BlockSpec(memory_space=pltpu.SMEM),
        out_shape=jax.ShapeDtypeStruct((1, 1), jnp.float32),
        scratch_shapes=[pltpu.SMEM((8,), jnp.float32)],
    )(epoch_arr, sem_logits, geo_logits, target2d, feat_flat, prototypes,
      aff_flat, d2s, d2i, bdy_flat)
    return out[0, 0]


def kernel(sem_logits, geo_logits, sem_feat_dense, affinity, prototypes,
           input_jafar_feat, bdy_logits, target, k_idx, epoch):
    B, N, C = sem_feat_dense.shape
    K = k_idx.shape[-1]
    D = input_jafar_feat.shape[-1]
    BN = B * N

    feat_flat = sem_feat_dense.reshape(BN, C)
    inp_flat = input_jafar_feat.reshape(BN, D)
    batch_offset = (jnp.arange(B, dtype=k_idx.dtype) * N)[:, None, None]
    idx_flat = (k_idx + batch_offset).reshape(BN * K)

    nbr_feat = jnp.take(feat_flat, idx_flat, axis=0).reshape(BN, K, C)
    d2s = jnp.sum((feat_flat[:, None, :] - nbr_feat) ** 2, axis=-1)
    nbr_inp = jnp.take(inp_flat, idx_flat, axis=0).reshape(BN, K, D)
    d2i = jnp.sum((inp_flat[:, None, :] - nbr_inp) ** 2, axis=-1)

    epoch_arr = jnp.asarray(epoch, dtype=jnp.int32).reshape(1)
    target2d = target.reshape(BN, 1)
    bdy_flat = bdy_logits.reshape(BN, 1)
    aff_flat = affinity.reshape(BN, K)

    return _dense_loss(epoch_arr, sem_logits, geo_logits, target2d,
                       feat_flat, prototypes, aff_flat, d2s, d2i, bdy_flat)


# speedup vs baseline: 2.4906x; 2.4906x over previous
import functools
import math

import jax
import jax.numpy as jnp
from jax import lax
from jax.experimental import pallas as pl
from jax.experimental.pallas import tpu as pltpu
from jax.experimental.pallas import tpu_sc as plsc

LAMBDA_SUP = 10.0
LAMBDA_CON = 1.0
LAMBDA_AFF = 0.1
LAMBDA_DIST = 0.1
LAMBDA_BDY = 0.5
WARMUP_EPOCHS = 15
IGNORE_INDEX = 255

BLK = 4096


def _dense_loss_kernel(epoch_ref, sem_ref, geo_ref, tgt_ref, feat_ref,
                       proto_ref, aff_ref, d2s_ref, d2i_ref, bdy_ref,
                       out_ref, acc_ref):
    i = pl.program_id(0)
    nsteps = pl.num_programs(0)
    blk, NCLS = sem_ref.shape
    C = feat_ref.shape[1]
    K = aff_ref.shape[1]

    tgt = tgt_ref[...]
    valid = (tgt != IGNORE_INDEX)
    validf = valid.astype(jnp.float32)
    nvalid = jnp.sum(validf)
    cls_iota = jax.lax.broadcasted_iota(jnp.int32, (blk, NCLS), 1)
    onehot = (cls_iota == tgt).astype(jnp.float32)

    def softmax_parts(x):
        m = jnp.max(x, axis=1, keepdims=True)
        e = jnp.exp(x - m)
        s = jnp.sum(e, axis=1, keepdims=True)
        lse = jnp.log(s) + m
        p = e / s
        return p, lse

    sem = sem_ref[...]
    geo = geo_ref[...]
    p_sem, lse_sem = softmax_parts(sem)
    p_geo, lse_geo = softmax_parts(geo)
    nll_sem = (lse_sem[:, 0] - jnp.sum(sem * onehot, axis=1)) * validf[:, 0]
    nll_geo = (lse_geo[:, 0] - jnp.sum(geo * onehot, axis=1)) * validf[:, 0]
    nll_sum = jnp.sum(nll_sem) + jnp.sum(nll_geo)

    eps = 1e-6
    pse = p_sem + eps
    pge = p_geo + eps
    log_pse = jnp.log(pse)
    log_pge = jnp.log(pge)
    kl_sg = jnp.sum(pge * (log_pge - log_pse))
    kl_gs = jnp.sum(pse * (log_pse - log_pge))

    aff = aff_ref[...]
    d2s = d2s_ref[...]
    amask = (aff > 0.8).astype(jnp.float32)
    aff_num = jnp.sum(aff * d2s * amask) * (1.0 / math.sqrt(C))
    mask_sum = jnp.sum(amask)

    feat = feat_ref[...]
    fnorm = jnp.maximum(jnp.sqrt(jnp.sum(feat * feat, axis=1, keepdims=True)),
                        1e-12)
    nf = feat / fnorm
    proto = proto_ref[...]
    pnorm = jnp.maximum(jnp.sqrt(jnp.sum(proto * proto, axis=1,
                                         keepdims=True)), 1e-12)
    nproto = proto / pnorm
    sim = jax.lax.dot_general(nf, nproto, (((1,), (1,)), ((), ())),
                              preferred_element_type=jnp.float32)
    tsim = jnp.sum(sim * onehot, axis=1)
    dist_sum = jnp.sum(validf[:, 0] * (1.0 - tsim))

    d2i = d2i_ref[...]
    jd = jnp.sqrt(d2i)
    es = jnp.sum(jd, axis=1) * (1.0 / K)
    tb = jax.nn.sigmoid((es - 0.15) * 20.0)
    x = bdy_ref[...][:, 0]
    bce = jnp.maximum(x, 0.0) - x * tb + jnp.log1p(jnp.exp(-jnp.abs(x)))
    bce_sum = jnp.sum(bce)

    @pl.when(i == 0)
    def _init():
        for j in range(8):
            acc_ref[j] = 0.0

    acc_ref[0] += nll_sum
    acc_ref[1] += nvalid
    acc_ref[2] += kl_sg
    acc_ref[3] += kl_gs
    acc_ref[4] += aff_num
    acc_ref[5] += mask_sum
    acc_ref[6] += dist_sum
    acc_ref[7] += bce_sum

    @pl.when(i == nsteps - 1)
    def _finalize():
        BN = blk * nsteps
        nv = jnp.maximum(acc_ref[1], 1.0)
        loss_sup = acc_ref[0] / nv
        epoch = epoch_ref[0]
        in_warmup = epoch < WARMUP_EPOCHS
        progress = jnp.clip(
            (epoch.astype(jnp.float32) - 1.0) / WARMUP_EPOCHS, 0.0, 1.0)
        lam_con = jnp.where(in_warmup, LAMBDA_CON * progress * 0.1,
                            LAMBDA_CON)
        kl_sg_m = acc_ref[2] / BN
        kl_gs_m = acc_ref[3] / BN
        loss_con = jnp.where(in_warmup, kl_sg_m, (kl_sg_m + kl_gs_m) * 0.5)
        loss_aff = acc_ref[4] / (acc_ref[5] + 1e-6)
        loss_dist = acc_ref[6] / nv
        loss_bdy = acc_ref[7] / BN
        out_ref[0, 0] = (loss_sup * LAMBDA_SUP + loss_con * lam_con
                         + loss_aff * LAMBDA_AFF + loss_dist * LAMBDA_DIST
                         + loss_bdy * LAMBDA_BDY)


def _dense_loss(epoch_arr, sem_logits, geo_logits, target2d, feat_flat,
                prototypes, aff_flat, d2s, d2i, bdy_flat):
    BN, NCLS = sem_logits.shape
    K = aff_flat.shape[1]
    C = feat_flat.shape[1]
    nsteps = BN // BLK
    out = pl.pallas_call(
        _dense_loss_kernel,
        grid=(nsteps,),
        in_specs=[
            pl.BlockSpec(memory_space=pltpu.SMEM),
            pl.BlockSpec((BLK, NCLS), lambda i: (i, 0)),
            pl.BlockSpec((BLK, NCLS), lambda i: (i, 0)),
            pl.BlockSpec((BLK, 1), lambda i: (i, 0)),
            pl.BlockSpec((BLK, C), lambda i: (i, 0)),
            pl.BlockSpec((prototypes.shape[0], C), lambda i: (0, 0)),
            pl.BlockSpec((BLK, K), lambda i: (i, 0)),
            pl.BlockSpec((BLK, K), lambda i: (i, 0)),
            pl.BlockSpec((BLK, K), lambda i: (i, 0)),
            pl.BlockSpec((BLK, 1), lambda i: (i, 0)),
        ],
        out_specs=pl.BlockSpec(memory_space=pltpu.SMEM),
        out_shape=jax.ShapeDtypeStruct((1, 1), jnp.float32),
        scratch_shapes=[pltpu.SMEM((8,), jnp.float32)],
    )(epoch_arr, sem_logits, geo_logits, target2d, feat_flat, prototypes,
      aff_flat, d2s, d2i, bdy_flat)
    return out[0, 0]


def _make_sc_dist2(BN, K, C, D, N):
    info = plsc.get_sparse_core_info()
    NC, NS = info.num_cores, info.num_subcores
    NW = NC * NS
    per_w = BN // NW
    CH = 16
    NCHUNK = per_w // CH
    E = CH * K
    mesh = plsc.VectorSubcoreMesh(core_axis_name="c", subcore_axis_name="s")

    @functools.partial(
        pl.kernel,
        mesh=mesh,
        out_type=[
            jax.ShapeDtypeStruct((BN, K), jnp.float32),
            jax.ShapeDtypeStruct((BN, K), jnp.float32),
        ],
        scratch_types=[
            pltpu.VMEM((E,), jnp.int32),
            pltpu.VMEM((E, C), jnp.float32),
            pltpu.VMEM((E, D), jnp.float32),
            pltpu.VMEM((CH, C), jnp.float32),
            pltpu.VMEM((CH, D), jnp.float32),
            pltpu.VMEM((CH, K), jnp.float32),
            pltpu.VMEM((CH, K), jnp.float32),
            pltpu.SemaphoreType.DMA,
            pltpu.SemaphoreType.DMA,
        ],
        compiler_params=pltpu.CompilerParams(needs_layout_passes=False,
                                             use_tc_tiling_on_sc=False),
    )
    def sc_kernel(feat_hbm, inp_hbm, kidx_hbm, d2s_hbm, d2i_hbm,
                  idx_v, nbrf_v, nbri_v, cenf_v, ceni_v, outs_v, outi_v,
                  sem_f, sem_i):
        wid = lax.axis_index("s") * NC + lax.axis_index("c")
        base_row = wid * per_w
        batch_base = (base_row // N) * N

        def chunk_body(ch, _):
            row0 = base_row + ch * CH
            e0 = row0 * K
            pltpu.sync_copy(kidx_hbm.at[pl.ds(e0, E)], idx_v)
            for j in range(E // 16):
                sl = pl.ds(j * 16, 16)
                idx_v[sl] = idx_v[sl] + batch_base
            gf = pltpu.async_copy(feat_hbm.at[idx_v], nbrf_v, sem_f)
            gi = pltpu.async_copy(inp_hbm.at[idx_v], nbri_v, sem_i)
            pltpu.sync_copy(feat_hbm.at[pl.ds(row0, CH), :], cenf_v)
            pltpu.sync_copy(inp_hbm.at[pl.ds(row0, CH), :], ceni_v)
            gf.wait()
            gi.wait()

            def center_body(i, _):
                ridx = lax.iota(jnp.int32, 16) + i * K
                acc = jnp.zeros((16,), jnp.float32)
                for j in range(C // 16):
                    cfv = cenf_v[i, pl.ds(j * 16, 16)]
                    for l in range(16):
                        col = jnp.full((16,), j * 16 + l, jnp.int32)
                        dv = plsc.load_gather(nbrf_v, [ridx, col]) - cfv[l]
                        acc = acc + dv * dv
                outs_v[i, :] = acc
                acc2 = jnp.zeros((16,), jnp.float32)
                for j in range(D // 16):
                    civ = ceni_v[i, pl.ds(j * 16, 16)]
                    for l in range(16):
                        col = jnp.full((16,), j * 16 + l, jnp.int32)
                        dv = plsc.load_gather(nbri_v, [ridx, col]) - civ[l]
                        acc2 = acc2 + dv * dv
                outi_v[i, :] = acc2
                return _

            lax.fori_loop(0, CH, center_body, None)
            pltpu.sync_copy(outs_v, d2s_hbm.at[pl.ds(row0, CH), :])
            pltpu.sync_copy(outi_v, d2i_hbm.at[pl.ds(row0, CH), :])
            return _

        lax.fori_loop(0, NCHUNK, chunk_body, None)

    return sc_kernel


def kernel(sem_logits, geo_logits, sem_feat_dense, affinity, prototypes,
           input_jafar_feat, bdy_logits, target, k_idx, epoch):
    B, N, C = sem_feat_dense.shape
    K = k_idx.shape[-1]
    D = input_jafar_feat.shape[-1]
    BN = B * N

    feat_flat = sem_feat_dense.reshape(BN, C)
    inp_flat = input_jafar_feat.reshape(BN, D)
    kidx_flat = k_idx.reshape(BN * K)

    sc_kernel = _make_sc_dist2(BN, K, C, D, N)
    d2s, d2i = sc_kernel(feat_flat, inp_flat, kidx_flat)

    epoch_arr = jnp.asarray(epoch, dtype=jnp.int32).reshape(1)
    target2d = target.reshape(BN, 1)
    bdy_flat = bdy_logits.reshape(BN, 1)
    aff_flat = affinity.reshape(BN, K)

    return _dense_loss(epoch_arr, sem_logits, geo_logits, target2d,
                       feat_flat, prototypes, aff_flat, d2s, d2i, bdy_flat)


# speedup vs baseline: 2.5820x; 1.0367x over previous
import functools
import math

import jax
import jax.numpy as jnp
from jax import lax
from jax.experimental import pallas as pl
from jax.experimental.pallas import tpu as pltpu
from jax.experimental.pallas import tpu_sc as plsc

LAMBDA_SUP = 10.0
LAMBDA_CON = 1.0
LAMBDA_AFF = 0.1
LAMBDA_DIST = 0.1
LAMBDA_BDY = 0.5
WARMUP_EPOCHS = 15
IGNORE_INDEX = 255

BLK = 4096


def _dense_loss_kernel(epoch_ref, sem_ref, geo_ref, tgt_ref, feat_ref,
                       proto_ref, aff_ref, d2s_ref, d2i_ref, bdy_ref,
                       out_ref, acc_ref):
    i = pl.program_id(0)
    nsteps = pl.num_programs(0)
    blk, NCLS = sem_ref.shape
    C = feat_ref.shape[1]
    K = aff_ref.shape[1]

    tgt = tgt_ref[...]
    valid = (tgt != IGNORE_INDEX)
    validf = valid.astype(jnp.float32)
    nvalid = jnp.sum(validf)
    cls_iota = jax.lax.broadcasted_iota(jnp.int32, (blk, NCLS), 1)
    onehot = (cls_iota == tgt).astype(jnp.float32)

    def softmax_parts(x):
        m = jnp.max(x, axis=1, keepdims=True)
        e = jnp.exp(x - m)
        s = jnp.sum(e, axis=1, keepdims=True)
        lse = jnp.log(s) + m
        p = e / s
        return p, lse

    sem = sem_ref[...]
    geo = geo_ref[...]
    p_sem, lse_sem = softmax_parts(sem)
    p_geo, lse_geo = softmax_parts(geo)
    nll_sem = (lse_sem[:, 0] - jnp.sum(sem * onehot, axis=1)) * validf[:, 0]
    nll_geo = (lse_geo[:, 0] - jnp.sum(geo * onehot, axis=1)) * validf[:, 0]
    nll_sum = jnp.sum(nll_sem) + jnp.sum(nll_geo)

    eps = 1e-6
    pse = p_sem + eps
    pge = p_geo + eps
    log_pse = jnp.log(pse)
    log_pge = jnp.log(pge)
    kl_sg = jnp.sum(pge * (log_pge - log_pse))
    kl_gs = jnp.sum(pse * (log_pse - log_pge))

    aff = aff_ref[...]
    d2s = d2s_ref[...]
    amask = (aff > 0.8).astype(jnp.float32)
    aff_num = jnp.sum(aff * d2s * amask) * (1.0 / math.sqrt(C))
    mask_sum = jnp.sum(amask)

    feat = feat_ref[...]
    fnorm = jnp.maximum(jnp.sqrt(jnp.sum(feat * feat, axis=1, keepdims=True)),
                        1e-12)
    nf = feat / fnorm
    proto = proto_ref[...]
    pnorm = jnp.maximum(jnp.sqrt(jnp.sum(proto * proto, axis=1,
                                         keepdims=True)), 1e-12)
    nproto = proto / pnorm
    sim = jax.lax.dot_general(nf, nproto, (((1,), (1,)), ((), ())),
                              preferred_element_type=jnp.float32)
    tsim = jnp.sum(sim * onehot, axis=1)
    dist_sum = jnp.sum(validf[:, 0] * (1.0 - tsim))

    d2i = d2i_ref[...]
    jd = jnp.sqrt(d2i)
    es = jnp.sum(jd, axis=1) * (1.0 / K)
    tb = jax.nn.sigmoid((es - 0.15) * 20.0)
    x = bdy_ref[...][:, 0]
    bce = jnp.maximum(x, 0.0) - x * tb + jnp.log1p(jnp.exp(-jnp.abs(x)))
    bce_sum = jnp.sum(bce)

    @pl.when(i == 0)
    def _init():
        for j in range(8):
            acc_ref[j] = 0.0

    acc_ref[0] += nll_sum
    acc_ref[1] += nvalid
    acc_ref[2] += kl_sg
    acc_ref[3] += kl_gs
    acc_ref[4] += aff_num
    acc_ref[5] += mask_sum
    acc_ref[6] += dist_sum
    acc_ref[7] += bce_sum

    @pl.when(i == nsteps - 1)
    def _finalize():
        BN = blk * nsteps
        nv = jnp.maximum(acc_ref[1], 1.0)
        loss_sup = acc_ref[0] / nv
        epoch = epoch_ref[0]
        in_warmup = epoch < WARMUP_EPOCHS
        progress = jnp.clip(
            (epoch.astype(jnp.float32) - 1.0) / WARMUP_EPOCHS, 0.0, 1.0)
        lam_con = jnp.where(in_warmup, LAMBDA_CON * progress * 0.1,
                            LAMBDA_CON)
        kl_sg_m = acc_ref[2] / BN
        kl_gs_m = acc_ref[3] / BN
        loss_con = jnp.where(in_warmup, kl_sg_m, (kl_sg_m + kl_gs_m) * 0.5)
        loss_aff = acc_ref[4] / (acc_ref[5] + 1e-6)
        loss_dist = acc_ref[6] / nv
        loss_bdy = acc_ref[7] / BN
        out_ref[0, 0] = (loss_sup * LAMBDA_SUP + loss_con * lam_con
                         + loss_aff * LAMBDA_AFF + loss_dist * LAMBDA_DIST
                         + loss_bdy * LAMBDA_BDY)


def _dense_loss(epoch_arr, sem_logits, geo_logits, target2d, feat_flat,
                prototypes, aff_flat, d2s, d2i, bdy_flat):
    BN, NCLS = sem_logits.shape
    K = aff_flat.shape[1]
    C = feat_flat.shape[1]
    nsteps = BN // BLK
    out = pl.pallas_call(
        _dense_loss_kernel,
        grid=(nsteps,),
        in_specs=[
            pl.BlockSpec(memory_space=pltpu.SMEM),
            pl.BlockSpec((BLK, NCLS), lambda i: (i, 0)),
            pl.BlockSpec((BLK, NCLS), lambda i: (i, 0)),
            pl.BlockSpec((BLK, 1), lambda i: (i, 0)),
            pl.BlockSpec((BLK, C), lambda i: (i, 0)),
            pl.BlockSpec((prototypes.shape[0], C), lambda i: (0, 0)),
            pl.BlockSpec((BLK, K), lambda i: (i, 0)),
            pl.BlockSpec((BLK, K), lambda i: (i, 0)),
            pl.BlockSpec((BLK, K), lambda i: (i, 0)),
            pl.BlockSpec((BLK, 1), lambda i: (i, 0)),
        ],
        out_specs=pl.BlockSpec(memory_space=pltpu.SMEM),
        out_shape=jax.ShapeDtypeStruct((1, 1), jnp.float32),
        scratch_shapes=[pltpu.SMEM((8,), jnp.float32)],
    )(epoch_arr, sem_logits, geo_logits, target2d, feat_flat, prototypes,
      aff_flat, d2s, d2i, bdy_flat)
    return out[0, 0]


def _make_sc_dist2(BN, K, C, D, N):
    info = plsc.get_sparse_core_info()
    NC, NS = info.num_cores, info.num_subcores
    NW = NC * NS
    per_w = BN // NW
    CH = 16
    NCHUNK = per_w // CH
    E = CH * K
    mesh = plsc.VectorSubcoreMesh(core_axis_name="c", subcore_axis_name="s")

    @functools.partial(
        pl.kernel,
        mesh=mesh,
        out_type=[
            jax.ShapeDtypeStruct((BN, K), jnp.float32),
            jax.ShapeDtypeStruct((BN, K), jnp.float32),
        ],
        scratch_types=[
            pltpu.VMEM((E,), jnp.int32),
            pltpu.VMEM((E,), jnp.int32),
            pltpu.VMEM((E, C), jnp.float32),
            pltpu.VMEM((E, C), jnp.float32),
            pltpu.VMEM((E, D), jnp.float32),
            pltpu.VMEM((E, D), jnp.float32),
            pltpu.VMEM((CH, C), jnp.float32),
            pltpu.VMEM((CH, D), jnp.float32),
            pltpu.VMEM((CH, K), jnp.float32),
            pltpu.VMEM((CH, K), jnp.float32),
            pltpu.SemaphoreType.DMA,
            pltpu.SemaphoreType.DMA,
        ],
        compiler_params=pltpu.CompilerParams(needs_layout_passes=False,
                                             use_tc_tiling_on_sc=False),
    )
    def sc_kernel(feat_hbm, inp_hbm, kidx_hbm, d2s_hbm, d2i_hbm,
                  idx0_v, idx1_v, nbrf0_v, nbrf1_v, nbri0_v, nbri1_v,
                  cenf_v, ceni_v, outs_v, outi_v, sem0, sem1):
        wid = lax.axis_index("s") * NC + lax.axis_index("c")
        base_row = wid * per_w
        batch_base = (base_row // N) * N
        idx_bufs = (idx0_v, idx1_v)
        nbrf_bufs = (nbrf0_v, nbrf1_v)
        nbri_bufs = (nbri0_v, nbri1_v)
        sems = (sem0, sem1)
        NACC = 8

        def issue_gather(ch, slot):
            row0 = base_row + ch * CH
            idx_v = idx_bufs[slot]
            pltpu.sync_copy(kidx_hbm.at[pl.ds(row0 * K, E)], idx_v)
            for j in range(E // 16):
                sl = pl.ds(j * 16, 16)
                idx_v[sl] = idx_v[sl] + batch_base
            pltpu.async_copy(feat_hbm.at[idx_v], nbrf_bufs[slot], sems[slot])
            pltpu.async_copy(inp_hbm.at[idx_v], nbri_bufs[slot], sems[slot])

        def wait_gather(slot):
            pltpu.make_async_copy(feat_hbm.at[idx_bufs[slot]],
                                  nbrf_bufs[slot], sems[slot]).wait()
            pltpu.make_async_copy(inp_hbm.at[idx_bufs[slot]],
                                  nbri_bufs[slot], sems[slot]).wait()

        def compute_chunk(ch, slot):
            row0 = base_row + ch * CH
            nbrf_v = nbrf_bufs[slot]
            nbri_v = nbri_bufs[slot]
            pltpu.sync_copy(feat_hbm.at[pl.ds(row0, CH), :], cenf_v)
            pltpu.sync_copy(inp_hbm.at[pl.ds(row0, CH), :], ceni_v)
            wait_gather(slot)

            def center_body(i, _):
                ridx = lax.iota(jnp.int32, 16) + i * K
                accs = [jnp.zeros((16,), jnp.float32) for _ in range(NACC)]
                cf = [cenf_v[i, pl.ds(j * 16, 16)] for j in range(C // 16)]
                for c in range(C):
                    col = jnp.full((16,), c, jnp.int32)
                    dv = plsc.load_gather(nbrf_v, [ridx, col]) - cf[c // 16][c % 16]
                    accs[c % NACC] = accs[c % NACC] + dv * dv
                while len(accs) > 1:
                    accs = [a + b for a, b in zip(accs[::2], accs[1::2])]
                outs_v[i, :] = accs[0]
                acc2s = [jnp.zeros((16,), jnp.float32) for _ in range(NACC)]
                ci = [ceni_v[i, pl.ds(j * 16, 16)] for j in range(D // 16)]
                for c in range(D):
                    col = jnp.full((16,), c, jnp.int32)
                    dv = plsc.load_gather(nbri_v, [ridx, col]) - ci[c // 16][c % 16]
                    acc2s[c % NACC] = acc2s[c % NACC] + dv * dv
                while len(acc2s) > 1:
                    acc2s = [a + b for a, b in zip(acc2s[::2], acc2s[1::2])]
                outi_v[i, :] = acc2s[0]
                return _

            lax.fori_loop(0, CH, center_body, None)
            pltpu.sync_copy(outs_v, d2s_hbm.at[pl.ds(row0, CH), :])
            pltpu.sync_copy(outi_v, d2i_hbm.at[pl.ds(row0, CH), :])

        issue_gather(0, 0)

        def pair_body(h, _):
            ch0 = h * 2
            ch1 = ch0 + 1
            issue_gather(ch1, 1)
            compute_chunk(ch0, 0)

            @pl.when(ch1 + 1 < NCHUNK)
            def _():
                issue_gather(ch1 + 1, 0)

            compute_chunk(ch1, 1)
            return _

        lax.fori_loop(0, NCHUNK // 2, pair_body, None)

    return sc_kernel


def kernel(sem_logits, geo_logits, sem_feat_dense, affinity, prototypes,
           input_jafar_feat, bdy_logits, target, k_idx, epoch):
    B, N, C = sem_feat_dense.shape
    K = k_idx.shape[-1]
    D = input_jafar_feat.shape[-1]
    BN = B * N

    feat_flat = sem_feat_dense.reshape(BN, C)
    inp_flat = input_jafar_feat.reshape(BN, D)
    kidx_flat = k_idx.reshape(BN * K)

    sc_kernel = _make_sc_dist2(BN, K, C, D, N)
    d2s, d2i = sc_kernel(feat_flat, inp_flat, kidx_flat)

    epoch_arr = jnp.asarray(epoch, dtype=jnp.int32).reshape(1)
    target2d = target.reshape(BN, 1)
    bdy_flat = bdy_logits.reshape(BN, 1)
    aff_flat = affinity.reshape(BN, K)

    return _dense_loss(epoch_arr, sem_logits, geo_logits, target2d,
                       feat_flat, prototypes, aff_flat, d2s, d2i, bdy_flat)


# speedup vs baseline: 5.5068x; 2.1328x over previous
import functools
import math

import jax
import jax.numpy as jnp
from jax import lax
from jax.experimental import pallas as pl
from jax.experimental.pallas import tpu as pltpu
from jax.experimental.pallas import tpu_sc as plsc

LAMBDA_SUP = 10.0
LAMBDA_CON = 1.0
LAMBDA_AFF = 0.1
LAMBDA_DIST = 0.1
LAMBDA_BDY = 0.5
WARMUP_EPOCHS = 15
IGNORE_INDEX = 255

BLK = 4096


def _dense_loss_kernel(epoch_ref, sem_ref, geo_ref, tgt_ref, feat_ref,
                       proto_ref, aff_ref, d2s_ref, d2i_ref, bdy_ref,
                       out_ref, acc_ref):
    i = pl.program_id(0)
    nsteps = pl.num_programs(0)
    blk, NCLS = sem_ref.shape
    C = feat_ref.shape[1]
    K = aff_ref.shape[1]

    tgt = tgt_ref[...]
    valid = (tgt != IGNORE_INDEX)
    validf = valid.astype(jnp.float32)
    nvalid = jnp.sum(validf)
    cls_iota = jax.lax.broadcasted_iota(jnp.int32, (blk, NCLS), 1)
    onehot = (cls_iota == tgt).astype(jnp.float32)

    def softmax_parts(x):
        m = jnp.max(x, axis=1, keepdims=True)
        e = jnp.exp(x - m)
        s = jnp.sum(e, axis=1, keepdims=True)
        lse = jnp.log(s) + m
        p = e / s
        return p, lse

    sem = sem_ref[...]
    geo = geo_ref[...]
    p_sem, lse_sem = softmax_parts(sem)
    p_geo, lse_geo = softmax_parts(geo)
    nll_sem = (lse_sem[:, 0] - jnp.sum(sem * onehot, axis=1)) * validf[:, 0]
    nll_geo = (lse_geo[:, 0] - jnp.sum(geo * onehot, axis=1)) * validf[:, 0]
    nll_sum = jnp.sum(nll_sem) + jnp.sum(nll_geo)

    eps = 1e-6
    pse = p_sem + eps
    pge = p_geo + eps
    log_pse = jnp.log(pse)
    log_pge = jnp.log(pge)
    kl_sg = jnp.sum(pge * (log_pge - log_pse))
    kl_gs = jnp.sum(pse * (log_pse - log_pge))

    aff = aff_ref[...]
    d2s = d2s_ref[...]
    amask = (aff > 0.8).astype(jnp.float32)
    aff_num = jnp.sum(aff * d2s * amask) * (1.0 / math.sqrt(C))
    mask_sum = jnp.sum(amask)

    feat = feat_ref[...]
    fnorm = jnp.maximum(jnp.sqrt(jnp.sum(feat * feat, axis=1, keepdims=True)),
                        1e-12)
    nf = feat / fnorm
    proto = proto_ref[...]
    pnorm = jnp.maximum(jnp.sqrt(jnp.sum(proto * proto, axis=1,
                                         keepdims=True)), 1e-12)
    nproto = proto / pnorm
    sim = jax.lax.dot_general(nf, nproto, (((1,), (1,)), ((), ())),
                              preferred_element_type=jnp.float32)
    tsim = jnp.sum(sim * onehot, axis=1)
    dist_sum = jnp.sum(validf[:, 0] * (1.0 - tsim))

    d2i = d2i_ref[...]
    jd = jnp.sqrt(d2i)
    es = jnp.sum(jd, axis=1) * (1.0 / K)
    tb = jax.nn.sigmoid((es - 0.15) * 20.0)
    x = bdy_ref[...][:, 0]
    bce = jnp.maximum(x, 0.0) - x * tb + jnp.log1p(jnp.exp(-jnp.abs(x)))
    bce_sum = jnp.sum(bce)

    @pl.when(i == 0)
    def _init():
        for j in range(8):
            acc_ref[j] = 0.0

    acc_ref[0] += nll_sum
    acc_ref[1] += nvalid
    acc_ref[2] += kl_sg
    acc_ref[3] += kl_gs
    acc_ref[4] += aff_num
    acc_ref[5] += mask_sum
    acc_ref[6] += dist_sum
    acc_ref[7] += bce_sum

    @pl.when(i == nsteps - 1)
    def _finalize():
        BN = blk * nsteps
        nv = jnp.maximum(acc_ref[1], 1.0)
        loss_sup = acc_ref[0] / nv
        epoch = epoch_ref[0]
        in_warmup = epoch < WARMUP_EPOCHS
        progress = jnp.clip(
            (epoch.astype(jnp.float32) - 1.0) / WARMUP_EPOCHS, 0.0, 1.0)
        lam_con = jnp.where(in_warmup, LAMBDA_CON * progress * 0.1,
                            LAMBDA_CON)
        kl_sg_m = acc_ref[2] / BN
        kl_gs_m = acc_ref[3] / BN
        loss_con = jnp.where(in_warmup, kl_sg_m, (kl_sg_m + kl_gs_m) * 0.5)
        loss_aff = acc_ref[4] / (acc_ref[5] + 1e-6)
        loss_dist = acc_ref[6] / nv
        loss_bdy = acc_ref[7] / BN
        out_ref[0, 0] = (loss_sup * LAMBDA_SUP + loss_con * lam_con
                         + loss_aff * LAMBDA_AFF + loss_dist * LAMBDA_DIST
                         + loss_bdy * LAMBDA_BDY)


def _dense_loss(epoch_arr, sem_logits, geo_logits, target2d, feat_flat,
                prototypes, aff_flat, d2s, d2i, bdy_flat):
    BN, NCLS = sem_logits.shape
    K = aff_flat.shape[1]
    C = feat_flat.shape[1]
    nsteps = BN // BLK
    out = pl.pallas_call(
        _dense_loss_kernel,
        grid=(nsteps,),
        in_specs=[
            pl.BlockSpec(memory_space=pltpu.SMEM),
            pl.BlockSpec((BLK, NCLS), lambda i: (i, 0)),
            pl.BlockSpec((BLK, NCLS), lambda i: (i, 0)),
            pl.BlockSpec((BLK, 1), lambda i: (i, 0)),
            pl.BlockSpec((BLK, C), lambda i: (i, 0)),
            pl.BlockSpec((prototypes.shape[0], C), lambda i: (0, 0)),
            pl.BlockSpec((BLK, K), lambda i: (i, 0)),
            pl.BlockSpec((BLK, K), lambda i: (i, 0)),
            pl.BlockSpec((BLK, K), lambda i: (i, 0)),
            pl.BlockSpec((BLK, 1), lambda i: (i, 0)),
        ],
        out_specs=pl.BlockSpec(memory_space=pltpu.SMEM),
        out_shape=jax.ShapeDtypeStruct((1, 1), jnp.float32),
        scratch_shapes=[pltpu.SMEM((8,), jnp.float32)],
    )(epoch_arr, sem_logits, geo_logits, target2d, feat_flat, prototypes,
      aff_flat, d2s, d2i, bdy_flat)
    return out[0, 0]


def _make_sc_dist2(BN, K, C, D, N):
    info = plsc.get_sparse_core_info()
    NC, NS = info.num_cores, info.num_subcores
    NW = NC * NS
    per_w = BN // NW
    CH = 16
    NCHUNK = per_w // CH
    E = CH * K
    mesh = plsc.VectorSubcoreMesh(core_axis_name="c", subcore_axis_name="s")

    @functools.partial(
        pl.kernel,
        mesh=mesh,
        out_type=[
            jax.ShapeDtypeStruct((BN, K), jnp.float32),
            jax.ShapeDtypeStruct((BN, K), jnp.float32),
        ],
        scratch_types=[
            pltpu.VMEM((E,), jnp.int32),
            pltpu.VMEM((E,), jnp.int32),
            pltpu.VMEM((E, C), jnp.float32),
            pltpu.VMEM((E, C), jnp.float32),
            pltpu.VMEM((E, D), jnp.float32),
            pltpu.VMEM((E, D), jnp.float32),
            pltpu.VMEM((CH, C), jnp.float32),
            pltpu.VMEM((CH, D), jnp.float32),
            pltpu.VMEM((CH, C + 16), jnp.float32),
            pltpu.VMEM((CH, D + 16), jnp.float32),
            pltpu.VMEM((CH, K), jnp.float32),
            pltpu.VMEM((CH, K), jnp.float32),
            pltpu.SemaphoreType.DMA,
            pltpu.SemaphoreType.DMA,
        ],
        compiler_params=pltpu.CompilerParams(needs_layout_passes=False,
                                             use_tc_tiling_on_sc=False),
    )
    def sc_kernel(feat_hbm, inp_hbm, kidx_hbm, d2s_hbm, d2i_hbm,
                  idx0_v, idx1_v, nbrf0_v, nbrf1_v, nbri0_v, nbri1_v,
                  cenf_v, ceni_v, cenfd_v, cenid_v, outs_v, outi_v,
                  sem0, sem1):
        wid = lax.axis_index("s") * NC + lax.axis_index("c")
        base_row = wid * per_w
        batch_base = (base_row // N) * N
        idx_bufs = (idx0_v, idx1_v)
        nbrf_bufs = (nbrf0_v, nbrf1_v)
        nbri_bufs = (nbri0_v, nbri1_v)
        sems = (sem0, sem1)
        NACC = 8

        def issue_gather(ch, slot):
            row0 = base_row + ch * CH
            idx_v = idx_bufs[slot]
            pltpu.sync_copy(kidx_hbm.at[pl.ds(row0 * K, E)], idx_v)
            for j in range(E // 16):
                sl = pl.ds(j * 16, 16)
                idx_v[sl] = idx_v[sl] + batch_base
            pltpu.async_copy(feat_hbm.at[idx_v], nbrf_bufs[slot], sems[slot])
            pltpu.async_copy(inp_hbm.at[idx_v], nbri_bufs[slot], sems[slot])

        def wait_gather(slot):
            pltpu.make_async_copy(feat_hbm.at[idx_bufs[slot]],
                                  nbrf_bufs[slot], sems[slot]).wait()
            pltpu.make_async_copy(inp_hbm.at[idx_bufs[slot]],
                                  nbri_bufs[slot], sems[slot]).wait()

        def compute_chunk(ch, slot):
            row0 = base_row + ch * CH
            nbrf_v = nbrf_bufs[slot]
            nbri_v = nbri_bufs[slot]
            pltpu.sync_copy(feat_hbm.at[pl.ds(row0, CH), :], cenf_v)
            pltpu.sync_copy(inp_hbm.at[pl.ds(row0, CH), :], ceni_v)
            wait_gather(slot)
            lane = lax.iota(jnp.int32, 16)

            def dup_body(i, _):
                for j in range(C // 16):
                    cenfd_v[i, pl.ds(j * 16, 16)] = cenf_v[i, pl.ds(j * 16, 16)]
                cenfd_v[i, pl.ds(C, 16)] = cenf_v[i, pl.ds(0, 16)]
                for j in range(D // 16):
                    cenid_v[i, pl.ds(j * 16, 16)] = ceni_v[i, pl.ds(j * 16, 16)]
                cenid_v[i, pl.ds(D, 16)] = ceni_v[i, pl.ds(0, 16)]
                return _

            lax.fori_loop(0, CH, dup_body, None)

            def center_body(i, _):
                ridx = lane + i * K
                accs = [jnp.zeros((16,), jnp.float32) for _ in range(NACC)]
                for c in range(C):
                    col = (lane + c) & (C - 1)
                    cw = cenfd_v[i, pl.ds(c, 16)]
                    dv = plsc.load_gather(nbrf_v, [ridx, col]) - cw
                    accs[c % NACC] = accs[c % NACC] + dv * dv
                while len(accs) > 1:
                    accs = [a + b for a, b in zip(accs[::2], accs[1::2])]
                outs_v[i, :] = accs[0]
                acc2s = [jnp.zeros((16,), jnp.float32) for _ in range(NACC)]
                for c in range(D):
                    col = (lane + c) & (D - 1)
                    cw = cenid_v[i, pl.ds(c, 16)]
                    dv = plsc.load_gather(nbri_v, [ridx, col]) - cw
                    acc2s[c % NACC] = acc2s[c % NACC] + dv * dv
                while len(acc2s) > 1:
                    acc2s = [a + b for a, b in zip(acc2s[::2], acc2s[1::2])]
                outi_v[i, :] = acc2s[0]
                return _

            lax.fori_loop(0, CH, center_body, None)
            pltpu.sync_copy(outs_v, d2s_hbm.at[pl.ds(row0, CH), :])
            pltpu.sync_copy(outi_v, d2i_hbm.at[pl.ds(row0, CH), :])

        issue_gather(0, 0)

        def pair_body(h, _):
            ch0 = h * 2
            ch1 = ch0 + 1
            issue_gather(ch1, 1)
            compute_chunk(ch0, 0)

            @pl.when(ch1 + 1 < NCHUNK)
            def _():
                issue_gather(ch1 + 1, 0)

            compute_chunk(ch1, 1)
            return _

        lax.fori_loop(0, NCHUNK // 2, pair_body, None)

    return sc_kernel


def kernel(sem_logits, geo_logits, sem_feat_dense, affinity, prototypes,
           input_jafar_feat, bdy_logits, target, k_idx, epoch):
    B, N, C = sem_feat_dense.shape
    K = k_idx.shape[-1]
    D = input_jafar_feat.shape[-1]
    BN = B * N

    feat_flat = sem_feat_dense.reshape(BN, C)
    inp_flat = input_jafar_feat.reshape(BN, D)
    kidx_flat = k_idx.reshape(BN * K)

    sc_kernel = _make_sc_dist2(BN, K, C, D, N)
    d2s, d2i = sc_kernel(feat_flat, inp_flat, kidx_flat)

    epoch_arr = jnp.asarray(epoch, dtype=jnp.int32).reshape(1)
    target2d = target.reshape(BN, 1)
    bdy_flat = bdy_logits.reshape(BN, 1)
    aff_flat = affinity.reshape(BN, K)

    return _dense_loss(epoch_arr, sem_logits, geo_logits, target2d,
                       feat_flat, prototypes, aff_flat, d2s, d2i, bdy_flat)


# speedup vs baseline: 6.8443x; 1.2429x over previous
import functools
import math

import jax
import jax.numpy as jnp
from jax import lax
from jax.experimental import pallas as pl
from jax.experimental.pallas import tpu as pltpu
from jax.experimental.pallas import tpu_sc as plsc

LAMBDA_SUP = 10.0
LAMBDA_CON = 1.0
LAMBDA_AFF = 0.1
LAMBDA_DIST = 0.1
LAMBDA_BDY = 0.5
WARMUP_EPOCHS = 15
IGNORE_INDEX = 255

BLK = 4096


def _dense_loss_kernel(epoch_ref, sem_ref, geo_ref, tgt_ref, feat_ref,
                       proto_ref, aff_ref, d2s_ref, d2i_ref, bdy_ref,
                       out_ref, acc_ref):
    i = pl.program_id(0)
    nsteps = pl.num_programs(0)
    blk, NCLS = sem_ref.shape
    C = feat_ref.shape[1]
    K = aff_ref.shape[1]

    tgt = tgt_ref[...]
    valid = (tgt != IGNORE_INDEX)
    validf = valid.astype(jnp.float32)
    nvalid = jnp.sum(validf)
    cls_iota = jax.lax.broadcasted_iota(jnp.int32, (blk, NCLS), 1)
    onehot = (cls_iota == tgt).astype(jnp.float32)

    def softmax_parts(x):
        m = jnp.max(x, axis=1, keepdims=True)
        e = jnp.exp(x - m)
        s = jnp.sum(e, axis=1, keepdims=True)
        lse = jnp.log(s) + m
        p = e / s
        return p, lse

    sem = sem_ref[...]
    geo = geo_ref[...]
    p_sem, lse_sem = softmax_parts(sem)
    p_geo, lse_geo = softmax_parts(geo)
    nll_sem = (lse_sem[:, 0] - jnp.sum(sem * onehot, axis=1)) * validf[:, 0]
    nll_geo = (lse_geo[:, 0] - jnp.sum(geo * onehot, axis=1)) * validf[:, 0]
    nll_sum = jnp.sum(nll_sem) + jnp.sum(nll_geo)

    eps = 1e-6
    pse = p_sem + eps
    pge = p_geo + eps
    log_pse = jnp.log(pse)
    log_pge = jnp.log(pge)
    kl_sg = jnp.sum(pge * (log_pge - log_pse))
    kl_gs = jnp.sum(pse * (log_pse - log_pge))

    aff = aff_ref[...]
    d2s = d2s_ref[...]
    amask = (aff > 0.8).astype(jnp.float32)
    aff_num = jnp.sum(aff * d2s * amask) * (1.0 / math.sqrt(C))
    mask_sum = jnp.sum(amask)

    feat = feat_ref[...]
    fnorm = jnp.maximum(jnp.sqrt(jnp.sum(feat * feat, axis=1, keepdims=True)),
                        1e-12)
    nf = feat / fnorm
    proto = proto_ref[...]
    pnorm = jnp.maximum(jnp.sqrt(jnp.sum(proto * proto, axis=1,
                                         keepdims=True)), 1e-12)
    nproto = proto / pnorm
    sim = jax.lax.dot_general(nf, nproto, (((1,), (1,)), ((), ())),
                              preferred_element_type=jnp.float32)
    tsim = jnp.sum(sim * onehot, axis=1)
    dist_sum = jnp.sum(validf[:, 0] * (1.0 - tsim))

    d2i = d2i_ref[...]
    jd = jnp.sqrt(d2i)
    es = jnp.sum(jd, axis=1) * (1.0 / K)
    tb = jax.nn.sigmoid((es - 0.15) * 20.0)
    x = bdy_ref[...][:, 0]
    bce = jnp.maximum(x, 0.0) - x * tb + jnp.log1p(jnp.exp(-jnp.abs(x)))
    bce_sum = jnp.sum(bce)

    @pl.when(i == 0)
    def _init():
        for j in range(8):
            acc_ref[j] = 0.0

    acc_ref[0] += nll_sum
    acc_ref[1] += nvalid
    acc_ref[2] += kl_sg
    acc_ref[3] += kl_gs
    acc_ref[4] += aff_num
    acc_ref[5] += mask_sum
    acc_ref[6] += dist_sum
    acc_ref[7] += bce_sum

    @pl.when(i == nsteps - 1)
    def _finalize():
        BN = blk * nsteps
        nv = jnp.maximum(acc_ref[1], 1.0)
        loss_sup = acc_ref[0] / nv
        epoch = epoch_ref[0]
        in_warmup = epoch < WARMUP_EPOCHS
        progress = jnp.clip(
            (epoch.astype(jnp.float32) - 1.0) / WARMUP_EPOCHS, 0.0, 1.0)
        lam_con = jnp.where(in_warmup, LAMBDA_CON * progress * 0.1,
                            LAMBDA_CON)
        kl_sg_m = acc_ref[2] / BN
        kl_gs_m = acc_ref[3] / BN
        loss_con = jnp.where(in_warmup, kl_sg_m, (kl_sg_m + kl_gs_m) * 0.5)
        loss_aff = acc_ref[4] / (acc_ref[5] + 1e-6)
        loss_dist = acc_ref[6] / nv
        loss_bdy = acc_ref[7] / BN
        out_ref[0, 0] = (loss_sup * LAMBDA_SUP + loss_con * lam_con
                         + loss_aff * LAMBDA_AFF + loss_dist * LAMBDA_DIST
                         + loss_bdy * LAMBDA_BDY)


def _dense_loss(epoch_arr, sem_logits, geo_logits, target2d, feat_flat,
                prototypes, aff_flat, d2s, d2i, bdy_flat):
    BN, NCLS = sem_logits.shape
    K = aff_flat.shape[1]
    C = feat_flat.shape[1]
    nsteps = BN // BLK
    out = pl.pallas_call(
        _dense_loss_kernel,
        grid=(nsteps,),
        in_specs=[
            pl.BlockSpec(memory_space=pltpu.SMEM),
            pl.BlockSpec((BLK, NCLS), lambda i: (i, 0)),
            pl.BlockSpec((BLK, NCLS), lambda i: (i, 0)),
            pl.BlockSpec((BLK, 1), lambda i: (i, 0)),
            pl.BlockSpec((BLK, C), lambda i: (i, 0)),
            pl.BlockSpec((prototypes.shape[0], C), lambda i: (0, 0)),
            pl.BlockSpec((BLK, K), lambda i: (i, 0)),
            pl.BlockSpec((BLK, K), lambda i: (i, 0)),
            pl.BlockSpec((BLK, K), lambda i: (i, 0)),
            pl.BlockSpec((BLK, 1), lambda i: (i, 0)),
        ],
        out_specs=pl.BlockSpec(memory_space=pltpu.SMEM),
        out_shape=jax.ShapeDtypeStruct((1, 1), jnp.float32),
        scratch_shapes=[pltpu.SMEM((8,), jnp.float32)],
    )(epoch_arr, sem_logits, geo_logits, target2d, feat_flat, prototypes,
      aff_flat, d2s, d2i, bdy_flat)
    return out[0, 0]


def _make_sc_dist2(BN, K, C, D, N):
    info = plsc.get_sparse_core_info()
    NC, NS = info.num_cores, info.num_subcores
    NW = NC * NS
    per_w = BN // NW
    CH = 16
    NCHUNK = per_w // CH
    E = CH * K
    mesh = plsc.VectorSubcoreMesh(core_axis_name="c", subcore_axis_name="s")

    @functools.partial(
        pl.kernel,
        mesh=mesh,
        out_type=[
            jax.ShapeDtypeStruct((BN, K), jnp.float32),
            jax.ShapeDtypeStruct((BN, K), jnp.float32),
        ],
        scratch_types=[
            pltpu.VMEM((E,), jnp.int32),
            pltpu.VMEM((E,), jnp.int32),
            pltpu.VMEM((E, C), jnp.float32),
            pltpu.VMEM((E, C), jnp.float32),
            pltpu.VMEM((E, D), jnp.float32),
            pltpu.VMEM((E, D), jnp.float32),
            pltpu.VMEM((CH, C), jnp.float32),
            pltpu.VMEM((CH, D), jnp.float32),
            pltpu.VMEM((CH, K), jnp.float32),
            pltpu.VMEM((CH, K), jnp.float32),
            pltpu.SemaphoreType.DMA,
            pltpu.SemaphoreType.DMA,
        ],
        compiler_params=pltpu.CompilerParams(needs_layout_passes=False,
                                             use_tc_tiling_on_sc=False),
    )
    def sc_kernel(feat_hbm, inp_hbm, kidx_hbm, d2s_hbm, d2i_hbm,
                  idx0_v, idx1_v, nbrf0_v, nbrf1_v, nbri0_v, nbri1_v,
                  cenf_v, ceni_v, outs_v, outi_v, sem0, sem1):
        wid = lax.axis_index("s") * NC + lax.axis_index("c")
        base_row = wid * per_w
        batch_base = (base_row // N) * N
        idx_bufs = (idx0_v, idx1_v)
        nbrf_bufs = (nbrf0_v, nbrf1_v)
        nbri_bufs = (nbri0_v, nbri1_v)
        sems = (sem0, sem1)
        NACC = 4

        def issue_gather(ch, slot):
            row0 = base_row + ch * CH
            idx_v = idx_bufs[slot]
            pltpu.sync_copy(kidx_hbm.at[pl.ds(row0 * K, E)], idx_v)
            for j in range(E // 16):
                sl = pl.ds(j * 16, 16)
                idx_v[sl] = idx_v[sl] + batch_base
            pltpu.async_copy(feat_hbm.at[idx_v], nbrf_bufs[slot], sems[slot])
            pltpu.async_copy(inp_hbm.at[idx_v], nbri_bufs[slot], sems[slot])

        def wait_gather(slot):
            pltpu.make_async_copy(feat_hbm.at[idx_bufs[slot]],
                                  nbrf_bufs[slot], sems[slot]).wait()
            pltpu.make_async_copy(inp_hbm.at[idx_bufs[slot]],
                                  nbri_bufs[slot], sems[slot]).wait()

        def compute_chunk(ch, slot):
            row0 = base_row + ch * CH
            nbrf_v = nbrf_bufs[slot]
            nbri_v = nbri_bufs[slot]
            pltpu.sync_copy(feat_hbm.at[pl.ds(row0, CH), :], cenf_v)
            pltpu.sync_copy(inp_hbm.at[pl.ds(row0, CH), :], ceni_v)
            wait_gather(slot)
            lane = lax.iota(jnp.int32, 16)

            def center_body(i, _):
                ridx = lane + i * K
                accs = [jnp.zeros((16,), jnp.float32) for _ in range(NACC)]
                for g in range(C // 16):
                    cfg = cenf_v[i, pl.ds(g * 16, 16)]
                    for s in range(16):
                        rot = (lane + s) & 15
                        col = rot + (g * 16)
                        dv = (plsc.load_gather(nbrf_v, [ridx, col])
                              - jnp.take_along_axis(cfg, rot, axis=0))
                        accs[s % NACC] = accs[s % NACC] + dv * dv
                while len(accs) > 1:
                    accs = [a + b for a, b in zip(accs[::2], accs[1::2])]
                outs_v[i, :] = accs[0]
                acc2s = [jnp.zeros((16,), jnp.float32) for _ in range(NACC)]
                for g in range(D // 16):
                    cig = ceni_v[i, pl.ds(g * 16, 16)]
                    for s in range(16):
                        rot = (lane + s) & 15
                        col = rot + (g * 16)
                        dv = (plsc.load_gather(nbri_v, [ridx, col])
                              - jnp.take_along_axis(cig, rot, axis=0))
                        acc2s[s % NACC] = acc2s[s % NACC] + dv * dv
                while len(acc2s) > 1:
                    acc2s = [a + b for a, b in zip(acc2s[::2], acc2s[1::2])]
                outi_v[i, :] = acc2s[0]
                return _

            lax.fori_loop(0, CH, center_body, None)
            pltpu.sync_copy(outs_v, d2s_hbm.at[pl.ds(row0, CH), :])
            pltpu.sync_copy(outi_v, d2i_hbm.at[pl.ds(row0, CH), :])

        issue_gather(0, 0)

        def pair_body(h, _):
            ch0 = h * 2
            ch1 = ch0 + 1
            issue_gather(ch1, 1)
            compute_chunk(ch0, 0)

            @pl.when(ch1 + 1 < NCHUNK)
            def _():
                issue_gather(ch1 + 1, 0)

            compute_chunk(ch1, 1)
            return _

        lax.fori_loop(0, NCHUNK // 2, pair_body, None)

    return sc_kernel


def kernel(sem_logits, geo_logits, sem_feat_dense, affinity, prototypes,
           input_jafar_feat, bdy_logits, target, k_idx, epoch):
    B, N, C = sem_feat_dense.shape
    K = k_idx.shape[-1]
    D = input_jafar_feat.shape[-1]
    BN = B * N

    feat_flat = sem_feat_dense.reshape(BN, C)
    inp_flat = input_jafar_feat.reshape(BN, D)
    kidx_flat = k_idx.reshape(BN * K)

    sc_kernel = _make_sc_dist2(BN, K, C, D, N)
    d2s, d2i = sc_kernel(feat_flat, inp_flat, kidx_flat)

    epoch_arr = jnp.asarray(epoch, dtype=jnp.int32).reshape(1)
    target2d = target.reshape(BN, 1)
    bdy_flat = bdy_logits.reshape(BN, 1)
    aff_flat = affinity.reshape(BN, K)

    return _dense_loss(epoch_arr, sem_logits, geo_logits, target2d,
                       feat_flat, prototypes, aff_flat, d2s, d2i, bdy_flat)


# speedup vs baseline: 7.5951x; 1.1097x over previous
import functools
import math

import jax
import jax.numpy as jnp
from jax import lax
from jax.experimental import pallas as pl
from jax.experimental.pallas import tpu as pltpu
from jax.experimental.pallas import tpu_sc as plsc

LAMBDA_SUP = 10.0
LAMBDA_CON = 1.0
LAMBDA_AFF = 0.1
LAMBDA_DIST = 0.1
LAMBDA_BDY = 0.5
WARMUP_EPOCHS = 15
IGNORE_INDEX = 255

BLK = 4096


def _dense_a_kernel(sem_ref, geo_ref, tgt_ref, feat_ref, proto_ref,
                    out_ref, acc_ref):
    i = pl.program_id(0)
    nsteps = pl.num_programs(0)
    blk, NCLS = sem_ref.shape

    tgt = tgt_ref[...]
    valid = (tgt != IGNORE_INDEX)
    validf = valid.astype(jnp.float32)
    nvalid = jnp.sum(validf)
    cls_iota = jax.lax.broadcasted_iota(jnp.int32, (blk, NCLS), 1)
    onehot = (cls_iota == tgt).astype(jnp.float32)

    def softmax_parts(x):
        m = jnp.max(x, axis=1, keepdims=True)
        e = jnp.exp(x - m)
        s = jnp.sum(e, axis=1, keepdims=True)
        lse = jnp.log(s) + m
        p = e / s
        return p, lse

    sem = sem_ref[...]
    geo = geo_ref[...]
    p_sem, lse_sem = softmax_parts(sem)
    p_geo, lse_geo = softmax_parts(geo)
    nll_sem = (lse_sem[:, 0] - jnp.sum(sem * onehot, axis=1)) * validf[:, 0]
    nll_geo = (lse_geo[:, 0] - jnp.sum(geo * onehot, axis=1)) * validf[:, 0]
    nll_sum = jnp.sum(nll_sem) + jnp.sum(nll_geo)

    eps = 1e-6
    pse = p_sem + eps
    pge = p_geo + eps
    log_pse = jnp.log(pse)
    log_pge = jnp.log(pge)
    kl_sg = jnp.sum(pge * (log_pge - log_pse))
    kl_gs = jnp.sum(pse * (log_pse - log_pge))

    feat = feat_ref[...]
    fnorm = jnp.maximum(jnp.sqrt(jnp.sum(feat * feat, axis=1, keepdims=True)),
                        1e-12)
    nf = feat / fnorm
    proto = proto_ref[...]
    pnorm = jnp.maximum(jnp.sqrt(jnp.sum(proto * proto, axis=1,
                                         keepdims=True)), 1e-12)
    nproto = proto / pnorm
    sim = jax.lax.dot_general(nf, nproto, (((1,), (1,)), ((), ())),
                              preferred_element_type=jnp.float32)
    tsim = jnp.sum(sim * onehot, axis=1)
    dist_sum = jnp.sum(validf[:, 0] * (1.0 - tsim))

    @pl.when(i == 0)
    def _init():
        for j in range(5):
            acc_ref[j] = 0.0

    acc_ref[0] += nll_sum
    acc_ref[1] += nvalid
    acc_ref[2] += kl_sg
    acc_ref[3] += kl_gs
    acc_ref[4] += dist_sum

    @pl.when(i == nsteps - 1)
    def _store():
        for j in range(5):
            out_ref[0, j] = acc_ref[j]


def _dense_a(sem_logits, geo_logits, target2d, feat_flat, prototypes):
    BN, NCLS = sem_logits.shape
    C = feat_flat.shape[1]
    nsteps = BN // BLK
    return pl.pallas_call(
        _dense_a_kernel,
        grid=(nsteps,),
        in_specs=[
            pl.BlockSpec((BLK, NCLS), lambda i: (i, 0)),
            pl.BlockSpec((BLK, NCLS), lambda i: (i, 0)),
            pl.BlockSpec((BLK, 1), lambda i: (i, 0)),
            pl.BlockSpec((BLK, C), lambda i: (i, 0)),
            pl.BlockSpec((prototypes.shape[0], C), lambda i: (0, 0)),
        ],
        out_specs=pl.BlockSpec(memory_space=pltpu.SMEM),
        out_shape=jax.ShapeDtypeStruct((1, 5), jnp.float32),
        scratch_shapes=[pltpu.SMEM((5,), jnp.float32)],
    )(sem_logits, geo_logits, target2d, feat_flat, prototypes)


def _dense_b_kernel(epoch_ref, parts_ref, aff_ref, d2s_ref, d2i_ref, bdy_ref,
                    out_ref, acc_ref):
    i = pl.program_id(0)
    nsteps = pl.num_programs(0)
    blk, K = aff_ref.shape
    C = 128

    aff = aff_ref[...]
    d2s = d2s_ref[...]
    amask = (aff > 0.8).astype(jnp.float32)
    aff_num = jnp.sum(aff * d2s * amask) * (1.0 / math.sqrt(C))
    mask_sum = jnp.sum(amask)

    d2i = d2i_ref[...]
    jd = jnp.sqrt(d2i)
    es = jnp.sum(jd, axis=1) * (1.0 / K)
    tb = jax.nn.sigmoid((es - 0.15) * 20.0)
    x = bdy_ref[...][:, 0]
    bce = jnp.maximum(x, 0.0) - x * tb + jnp.log1p(jnp.exp(-jnp.abs(x)))
    bce_sum = jnp.sum(bce)

    @pl.when(i == 0)
    def _init():
        for j in range(3):
            acc_ref[j] = 0.0

    acc_ref[0] += aff_num
    acc_ref[1] += mask_sum
    acc_ref[2] += bce_sum

    @pl.when(i == nsteps - 1)
    def _finalize():
        BN = blk * nsteps
        nv = jnp.maximum(parts_ref[0, 1], 1.0)
        loss_sup = parts_ref[0, 0] / nv
        epoch = epoch_ref[0]
        in_warmup = epoch < WARMUP_EPOCHS
        progress = jnp.clip(
            (epoch.astype(jnp.float32) - 1.0) / WARMUP_EPOCHS, 0.0, 1.0)
        lam_con = jnp.where(in_warmup, LAMBDA_CON * progress * 0.1,
                            LAMBDA_CON)
        kl_sg_m = parts_ref[0, 2] / BN
        kl_gs_m = parts_ref[0, 3] / BN
        loss_con = jnp.where(in_warmup, kl_sg_m, (kl_sg_m + kl_gs_m) * 0.5)
        loss_aff = acc_ref[0] / (acc_ref[1] + 1e-6)
        loss_dist = parts_ref[0, 4] / nv
        loss_bdy = acc_ref[2] / BN
        out_ref[0, 0] = (loss_sup * LAMBDA_SUP + loss_con * lam_con
                         + loss_aff * LAMBDA_AFF + loss_dist * LAMBDA_DIST
                         + loss_bdy * LAMBDA_BDY)


def _dense_b(epoch_arr, parts, aff_flat, d2s, d2i, bdy_flat):
    BN, K = aff_flat.shape
    nsteps = BN // BLK
    out = pl.pallas_call(
        _dense_b_kernel,
        grid=(nsteps,),
        in_specs=[
            pl.BlockSpec(memory_space=pltpu.SMEM),
            pl.BlockSpec(memory_space=pltpu.SMEM),
            pl.BlockSpec((BLK, K), lambda i: (i, 0)),
            pl.BlockSpec((BLK, K), lambda i: (i, 0)),
            pl.BlockSpec((BLK, K), lambda i: (i, 0)),
            pl.BlockSpec((BLK, 1), lambda i: (i, 0)),
        ],
        out_specs=pl.BlockSpec(memory_space=pltpu.SMEM),
        out_shape=jax.ShapeDtypeStruct((1, 1), jnp.float32),
        scratch_shapes=[pltpu.SMEM((3,), jnp.float32)],
    )(epoch_arr, parts, aff_flat, d2s, d2i, bdy_flat)
    return out[0, 0]


def _make_sc_dist2(BN, K, C, D, N):
    info = plsc.get_sparse_core_info()
    NC, NS = info.num_cores, info.num_subcores
    NW = NC * NS
    per_w = BN // NW
    CH = 16
    NCHUNK = per_w // CH
    E = CH * K
    mesh = plsc.VectorSubcoreMesh(core_axis_name="c", subcore_axis_name="s")

    @functools.partial(
        pl.kernel,
        mesh=mesh,
        out_type=[
            jax.ShapeDtypeStruct((BN, K), jnp.float32),
            jax.ShapeDtypeStruct((BN, K), jnp.float32),
        ],
        scratch_types=[
            pltpu.VMEM((E,), jnp.int32),
            pltpu.VMEM((E,), jnp.int32),
            pltpu.VMEM((E, C), jnp.float32),
            pltpu.VMEM((E, C), jnp.float32),
            pltpu.VMEM((E, D), jnp.float32),
            pltpu.VMEM((E, D), jnp.float32),
            pltpu.VMEM((CH, C), jnp.float32),
            pltpu.VMEM((CH, D), jnp.float32),
            pltpu.VMEM((CH, K), jnp.float32),
            pltpu.VMEM((CH, K), jnp.float32),
            pltpu.SemaphoreType.DMA,
            pltpu.SemaphoreType.DMA,
        ],
        compiler_params=pltpu.CompilerParams(needs_layout_passes=False,
                                             use_tc_tiling_on_sc=False),
    )
    def sc_kernel(feat_hbm, inp_hbm, kidx_hbm, d2s_hbm, d2i_hbm,
                  idx0_v, idx1_v, nbrf0_v, nbrf1_v, nbri0_v, nbri1_v,
                  cenf_v, ceni_v, outs_v, outi_v, sem0, sem1):
        wid = lax.axis_index("s") * NC + lax.axis_index("c")
        base_row = wid * per_w
        batch_base = (base_row // N) * N
        idx_bufs = (idx0_v, idx1_v)
        nbrf_bufs = (nbrf0_v, nbrf1_v)
        nbri_bufs = (nbri0_v, nbri1_v)
        sems = (sem0, sem1)
        NACC = 4

        def issue_gather(ch, slot):
            row0 = base_row + ch * CH
            idx_v = idx_bufs[slot]
            pltpu.sync_copy(kidx_hbm.at[pl.ds(row0 * K, E)], idx_v)
            for j in range(E // 16):
                sl = pl.ds(j * 16, 16)
                idx_v[sl] = idx_v[sl] + batch_base
            pltpu.async_copy(feat_hbm.at[idx_v], nbrf_bufs[slot], sems[slot])
            pltpu.async_copy(inp_hbm.at[idx_v], nbri_bufs[slot], sems[slot])

        def wait_gather(slot):
            pltpu.make_async_copy(feat_hbm.at[idx_bufs[slot]],
                                  nbrf_bufs[slot], sems[slot]).wait()
            pltpu.make_async_copy(inp_hbm.at[idx_bufs[slot]],
                                  nbri_bufs[slot], sems[slot]).wait()

        def compute_chunk(ch, slot):
            row0 = base_row + ch * CH
            nbrf_v = nbrf_bufs[slot]
            nbri_v = nbri_bufs[slot]
            pltpu.sync_copy(feat_hbm.at[pl.ds(row0, CH), :], cenf_v)
            pltpu.sync_copy(inp_hbm.at[pl.ds(row0, CH), :], ceni_v)
            wait_gather(slot)
            lane = lax.iota(jnp.int32, 16)

            def center_body(i, _):
                ridx = lane + i * K
                accs = [jnp.zeros((16,), jnp.float32) for _ in range(NACC)]
                for g in range(C // 16):
                    cfg = cenf_v[i, pl.ds(g * 16, 16)]
                    for s in range(16):
                        rot = (lane + s) & 15
                        col = rot + (g * 16)
                        dv = (plsc.load_gather(nbrf_v, [ridx, col])
                              - jnp.take_along_axis(cfg, rot, axis=0))
                        accs[s % NACC] = accs[s % NACC] + dv * dv
                while len(accs) > 1:
                    accs = [a + b for a, b in zip(accs[::2], accs[1::2])]
                outs_v[i, :] = accs[0]
                acc2s = [jnp.zeros((16,), jnp.float32) for _ in range(NACC)]
                for g in range(D // 16):
                    cig = ceni_v[i, pl.ds(g * 16, 16)]
                    for s in range(16):
                        rot = (lane + s) & 15
                        col = rot + (g * 16)
                        dv = (plsc.load_gather(nbri_v, [ridx, col])
                              - jnp.take_along_axis(cig, rot, axis=0))
                        acc2s[s % NACC] = acc2s[s % NACC] + dv * dv
                while len(acc2s) > 1:
                    acc2s = [a + b for a, b in zip(acc2s[::2], acc2s[1::2])]
                outi_v[i, :] = acc2s[0]
                return _

            lax.fori_loop(0, CH, center_body, None)
            pltpu.sync_copy(outs_v, d2s_hbm.at[pl.ds(row0, CH), :])
            pltpu.sync_copy(outi_v, d2i_hbm.at[pl.ds(row0, CH), :])

        issue_gather(0, 0)

        def pair_body(h, _):
            ch0 = h * 2
            ch1 = ch0 + 1
            issue_gather(ch1, 1)
            compute_chunk(ch0, 0)

            @pl.when(ch1 + 1 < NCHUNK)
            def _():
                issue_gather(ch1 + 1, 0)

            compute_chunk(ch1, 1)
            return _

        lax.fori_loop(0, NCHUNK // 2, pair_body, None)

    return sc_kernel


def kernel(sem_logits, geo_logits, sem_feat_dense, affinity, prototypes,
           input_jafar_feat, bdy_logits, target, k_idx, epoch):
    B, N, C = sem_feat_dense.shape
    K = k_idx.shape[-1]
    D = input_jafar_feat.shape[-1]
    BN = B * N

    feat_flat = sem_feat_dense.reshape(BN, C)
    inp_flat = input_jafar_feat.reshape(BN, D)
    kidx_flat = k_idx.reshape(BN * K)

    sc_kernel = _make_sc_dist2(BN, K, C, D, N)
    d2s, d2i = sc_kernel(feat_flat, inp_flat, kidx_flat)

    epoch_arr = jnp.asarray(epoch, dtype=jnp.int32).reshape(1)
    target2d = target.reshape(BN, 1)
    bdy_flat = bdy_logits.reshape(BN, 1)
    aff_flat = affinity.reshape(BN, K)

    parts = _dense_a(sem_logits, geo_logits, target2d, feat_flat, prototypes)
    return _dense_b(epoch_arr, parts, aff_flat, d2s, d2i, bdy_flat)
